# baseline (speedup 1.0000x reference)
import functools

import jax
import jax.numpy as jnp
import numpy as np
from jax.experimental import pallas as pl

_N_CH = 85
_FSIZE = 32
_NPIX = _FSIZE * _FSIZE
_STRIDE = 32.0
_W_SCALE = (116.0, 156.0, 373.0)
_H_SCALE = (90.0, 198.0, 326.0)


def _decode_kernel(x_ref, o_ref):
    a = pl.program_id(1)
    v = x_ref[0, 0]

    sig = jax.nn.sigmoid(v)
    ex = jnp.exp(v)

    shape3 = (_N_CH, _FSIZE, _FSIZE)
    row = jax.lax.broadcasted_iota(jnp.int32, shape3, 0)
    yi = jax.lax.broadcasted_iota(jnp.int32, (1, _FSIZE, _FSIZE), 1).astype(jnp.float32)
    xj = jax.lax.broadcasted_iota(jnp.int32, (1, _FSIZE, _FSIZE), 2).astype(jnp.float32)

    wa = jnp.where(a == 0, _W_SCALE[0], jnp.where(a == 1, _W_SCALE[1], _W_SCALE[2]))
    ha = jnp.where(a == 0, _H_SCALE[0], jnp.where(a == 1, _H_SCALE[1], _H_SCALE[2]))

    res = jnp.where(
        row == 0,
        (sig + xj) * _STRIDE,
        jnp.where(
            row == 1,
            (sig + yi) * _STRIDE,
            jnp.where(row == 2, ex * wa, jnp.where(row == 3, ex * ha, sig)),
        ),
    )
    t = jnp.transpose(res, (1, 2, 0))
    o_ref[0] = t.reshape(_NPIX, _N_CH)


@jax.jit
def kernel(xin):
    b = xin.shape[1]
    out = pl.pallas_call(
        _decode_kernel,
        grid=(b, 3),
        in_specs=[
            pl.BlockSpec(
                (1, 1, _N_CH, _FSIZE, _FSIZE), lambda i, j: (0, i, j, 0, 0)
            )
        ],
        out_specs=pl.BlockSpec((1, _NPIX, _N_CH), lambda i, j: (i, j, 0)),
        out_shape=jax.ShapeDtypeStruct((b, 3 * _NPIX, _N_CH), jnp.float32),
    )(xin)
    return out


# speedup vs baseline: 1.4442x; 1.4442x over previous
import functools

import jax
import jax.numpy as jnp
import numpy as np
from jax.experimental import pallas as pl

_N_CH = 85
_FSIZE = 32
_NPIX = _FSIZE * _FSIZE
_STRIDE = 32.0
_W_SCALE = (116.0, 156.0, 373.0)
_H_SCALE = (90.0, 198.0, 326.0)


def _decode_kernel(x_ref, o_ref):
    shape3 = (_N_CH, _FSIZE, _FSIZE)
    row = jax.lax.broadcasted_iota(jnp.int32, shape3, 0)
    yi = jax.lax.broadcasted_iota(jnp.int32, (1, _FSIZE, _FSIZE), 1).astype(jnp.float32)
    xj = jax.lax.broadcasted_iota(jnp.int32, (1, _FSIZE, _FSIZE), 2).astype(jnp.float32)

    for a in range(3):
        v = x_ref[0, a * _N_CH : (a + 1) * _N_CH]
        sig = jax.nn.sigmoid(v)
        ex = jnp.exp(v)
        res = jnp.where(
            row == 0,
            (sig + xj) * _STRIDE,
            jnp.where(
                row == 1,
                (sig + yi) * _STRIDE,
                jnp.where(
                    row == 2, ex * _W_SCALE[a], jnp.where(row == 3, ex * _H_SCALE[a], sig)
                ),
            ),
        )
        t = jnp.transpose(res, (1, 2, 0))
        o_ref[0, a * _NPIX : (a + 1) * _NPIX] = t.reshape(_NPIX, _N_CH)


@jax.jit
def kernel(xin):
    b = xin.shape[1]
    x = xin[0]
    out = pl.pallas_call(
        _decode_kernel,
        grid=(b,),
        in_specs=[pl.BlockSpec((1, 3 * _N_CH, _FSIZE, _FSIZE), lambda i: (i, 0, 0, 0))],
        out_specs=pl.BlockSpec((1, 3 * _NPIX, _N_CH), lambda i: (i, 0, 0)),
        out_shape=jax.ShapeDtypeStruct((b, 3 * _NPIX, _N_CH), jnp.float32),
    )(x)
    return out


# speedup vs baseline: 2.2522x; 1.5594x over previous
import functools

import jax
import jax.numpy as jnp
import numpy as np
from jax.experimental import pallas as pl

_N_CH = 85
_FSIZE = 32
_NPIX = _FSIZE * _FSIZE
_STRIDE = 32.0
_W_SCALE = (116.0, 156.0, 373.0)
_H_SCALE = (90.0, 198.0, 326.0)


def _decode_kernel(x_ref, o_ref):
    col = jax.lax.broadcasted_iota(jnp.int32, (1, _NPIX), 1)
    xj = (col % _FSIZE).astype(jnp.float32)
    yi = (col // _FSIZE).astype(jnp.float32)
    row = jax.lax.broadcasted_iota(jnp.int32, (_N_CH, _NPIX), 0)

    for a in range(3):
        va = x_ref[0, a * _N_CH : (a + 1) * _N_CH]
        sig = jax.nn.sigmoid(va)
        ex = jnp.exp(va)
        res = jnp.where(
            row == 0,
            (sig + xj) * _STRIDE,
            jnp.where(
                row == 1,
                (sig + yi) * _STRIDE,
                jnp.where(
                    row == 2,
                    ex * _W_SCALE[a],
                    jnp.where(row == 3, ex * _H_SCALE[a], sig),
                ),
            ),
        )
        o_ref[0, a * _NPIX : (a + 1) * _NPIX] = res.T


@jax.jit
def kernel(xin):
    b = xin.shape[1]
    x = xin[0].reshape(b, 3 * _N_CH, _NPIX)
    out = pl.pallas_call(
        _decode_kernel,
        grid=(b,),
        in_specs=[pl.BlockSpec((1, 3 * _N_CH, _NPIX), lambda i: (i, 0, 0))],
        out_specs=pl.BlockSpec((1, 3 * _NPIX, _N_CH), lambda i: (i, 0, 0)),
        out_shape=jax.ShapeDtypeStruct((b, 3 * _NPIX, _N_CH), jnp.float32),
    )(x)
    return out
